# Initial kernel scaffold; baseline (speedup 1.0000x reference)
import jax
import jax.numpy as jnp
from jax.experimental import pallas as pl


def kernel(R, Z, neighbor_idx, embeddings):
    raise NotImplementedError("write your pallas kernel here")


# speedup vs baseline: 229.6096x; 229.6096x over previous
import functools
import numpy as np
import jax
import jax.numpy as jnp
from jax import lax
from jax.experimental import pallas as pl
from jax.experimental.pallas import tpu as pltpu
from jax.experimental.pallas import tpu_sc as plsc

A = 10000
E = 320000
S = 119
NB = 7
NR = 5
R_MAX = 6.0
R_MIN = 0.5
NG = 20
NU = NR * NG
W = 104
AP = 10240
NC, NS, L = 2, 16, 16
NWK = NC * NS
CH = 128
CPW = 80
EP = NWK * CPW * CH
WE = 40
NF = 360

PAIRS2 = [(0, 0), (0, 1), (0, 2), (1, 1), (1, 2), (2, 2)]
TRIP3 = [(i, j, k) for i in range(3) for j in range(i, 3) for k in range(j, 3)]
T2 = [(r, s) for r in range(NR) for s in range(r, NR)]
T3 = [(r, s, t) for r in range(NR) for s in range(r, NR) for t in range(s, NR)]

BETTA = float(NB ** 2 / R_MAX ** 2)
RAD_NORM = float((2.0 * BETTA / np.pi) ** 0.25)
EMBED_NORM = float(1.0 / np.sqrt(NB))
SHIFTS = [float(np.float32(R_MIN + (R_MAX - R_MIN) / NB * b)) for b in range(NB)]
COSPOLY = [-9.451651683534124e-10, 0.007014570292085409, -0.038580119609832764,
           0.014410065487027168, 0.22450557351112366, 0.005562894977629185,
           -1.3372236490249634, 0.0004664007865358144, 4.058640003204346,
           6.792909061914543e-06, -4.934802532196045, 7.660506895490471e-09, 1.0]

_ROWS_PER_TILE = AP // NS


def _edge_kernel():
    mesh = plsc.VectorSubcoreMesh(core_axis_name="c", subcore_axis_name="s")

    @functools.partial(
        pl.kernel,
        out_type=jax.ShapeDtypeStruct((NC * AP, W), jnp.float32),
        mesh=mesh,
        compiler_params=pltpu.CompilerParams(needs_layout_passes=False,
                                             use_tc_tiling_on_sc=False),
        scratch_types=[
            pltpu.VMEM((A,), jnp.float32),
            pltpu.VMEM((A,), jnp.float32),
            pltpu.VMEM((A,), jnp.float32),
            pltpu.VMEM((A,), jnp.int32),
            pltpu.VMEM((CH,), jnp.int32),
            pltpu.VMEM((CH,), jnp.int32),
            pltpu.VMEM((CH,), jnp.int32),
            pltpu.VMEM((CH,), jnp.int32),
            pltpu.VMEM((CH, WE), jnp.float32),
            pltpu.VMEM((CH, W), jnp.float32),
            pltpu.VMEM_SHARED((AP, W), jnp.float32),
            pltpu.SemaphoreType.DMA,
        ],
    )
    def edge_kernel(gi_hbm, gj_hbm, seg_hbm, rx_hbm, ry_hbm, rz_hbm, z_hbm,
                    emb_hbm, zero_hbm, out_hbm,
                    rx_v, ry_v, rz_v, z_v, gi_v, gj_v, seg_v, pidx_v,
                    coeff_v, mrow_v, acc, sem):
        cid = lax.axis_index("c")
        sid = lax.axis_index("s")
        wid = sid * NC + cid

        pltpu.sync_copy(rx_hbm, rx_v)
        pltpu.sync_copy(ry_hbm, ry_v)
        pltpu.sync_copy(rz_hbm, rz_v)
        pltpu.sync_copy(z_hbm, z_v)

        pltpu.sync_copy(zero_hbm, acc.at[pl.ds(sid * _ROWS_PER_TILE, _ROWS_PER_TILE)])

        zeros16 = jnp.zeros((L,), jnp.float32)
        riota = lax.iota(jnp.int32, L)
        for g in range(CH // L):
            grows = g * L + riota
            for col in range(NU, W):
                plsc.store_scatter(mrow_v, [grows, jnp.full((L,), col, jnp.int32)], zeros16)

        plsc.subcore_barrier()

        def chunk_body(t, carry):
            base = pl.multiple_of((wid * CPW + t) * CH, CH)
            pltpu.sync_copy(gi_hbm.at[pl.ds(base, CH)], gi_v)
            pltpu.sync_copy(gj_hbm.at[pl.ds(base, CH)], gj_v)
            pltpu.sync_copy(seg_hbm.at[pl.ds(base, CH)], seg_v)

            for g in range(CH // L):
                ii = gi_v[pl.ds(g * L, L)]
                jj = gj_v[pl.ds(g * L, L)]
                zi = plsc.load_gather(z_v, [ii])
                zj = plsc.load_gather(z_v, [jj])
                pidx_v[pl.ds(g * L, L)] = zj * S + zi

            pltpu.async_copy(emb_hbm.at[pidx_v], coeff_v, sem).wait()

            for g in range(CH // L):
                grows = g * L + riota
                ii = gi_v[pl.ds(g * L, L)]
                jj = gj_v[pl.ds(g * L, L)]
                xi = plsc.load_gather(rx_v, [ii])
                yi = plsc.load_gather(ry_v, [ii])
                zi3 = plsc.load_gather(rz_v, [ii])
                xj = plsc.load_gather(rx_v, [jj])
                yj = plsc.load_gather(ry_v, [jj])
                zj3 = plsc.load_gather(rz_v, [jj])
                dx = xj - xi
                dy = yj - yi
                dz = zj3 - zi3
                r2 = dx * dx + dy * dy + dz * dz
                bits = plsc.bitcast(r2, jnp.int32)
                y = plsc.bitcast(jnp.full((L,), 0x5F3759DF, jnp.int32) - (bits >> 1),
                                 jnp.float32)
                for _ in range(3):
                    y = y * (1.5 - 0.5 * r2 * y * y)
                rinv = y
                dr = r2 * rinv
                nx = dx * rinv
                ny = dy * rinv
                nz = dz * rinv
                uu = jnp.minimum(dr, R_MAX) * (1.0 / R_MAX)
                cacc = jnp.full((L,), COSPOLY[0], jnp.float32)
                for cfl in COSPOLY[1:]:
                    cacc = cacc * uu + cfl
                cut = 0.5 * (cacc + 1.0)
                basis = []
                for b in range(NB):
                    tb = SHIFTS[b] - dr
                    basis.append(jnp.exp(-BETTA * (tb * tb)))
                scale = cut * (RAD_NORM * EMBED_NORM)
                rad = []
                for r in range(NR):
                    accr = None
                    for b in range(NB):
                        cvec = plsc.load_gather(coeff_v, [grows, jnp.full((L,), r * NB + b, jnp.int32)])
                        accr = cvec * basis[b] if accr is None else accr + cvec * basis[b]
                    rad.append(accr * scale)
                geom = [None] * NG
                geom[1], geom[2], geom[3] = nx, ny, nz
                for q, (a_, b_) in enumerate(PAIRS2):
                    geom[4 + q] = geom[1 + a_] * geom[1 + b_]
                for q, (a_, b_, c_) in enumerate(TRIP3):
                    geom[10 + q] = geom[4 + PAIRS2.index((a_, b_))] * geom[1 + c_]
                for r in range(NR):
                    plsc.store_scatter(mrow_v, [grows, jnp.full((L,), r * NG, jnp.int32)], rad[r])
                    for q in range(1, NG):
                        plsc.store_scatter(
                            mrow_v, [grows, jnp.full((L,), r * NG + q, jnp.int32)],
                            rad[r] * geom[q])

            pltpu.sync_copy(mrow_v, acc.at[seg_v], add=True)
            return carry

        lax.fori_loop(0, CPW, chunk_body, 0)
        plsc.subcore_barrier()

        row0 = sid * _ROWS_PER_TILE
        pltpu.sync_copy(acc.at[pl.ds(row0, _ROWS_PER_TILE)],
                        out_hbm.at[pl.ds(cid * AP + row0, _ROWS_PER_TILE)])

    return edge_kernel


def _contract_body(p_ref, o_ref):
    mu = {}
    for r in range(NR):
        for g in range(NG):
            c = r * NG + g
            mu[(r, g)] = p_ref[0, c] + p_ref[1, c]

    def m0c(r):
        return mu[(r, 0)]

    def m1c(r, i):
        return mu[(r, 1 + i)]

    def m2c(r, i, j):
        i, j = sorted((i, j))
        return mu[(r, 4 + PAIRS2.index((i, j)))]

    def m3c(r, i, j, k):
        i, j, k = sorted((i, j, k))
        return mu[(r, 10 + TRIP3.index((i, j, k)))]

    A2 = {(r, s): [[sum(m2c(r, i, j) * m2c(s, i, k) for i in range(3))
                    for k in range(3)] for j in range(3)] for (r, s) in T2}
    B5 = {(r, t): [sum(m1c(r, i) * m2c(t, i, j) for i in range(3))
                   for j in range(3)] for r in range(NR) for t in range(NR)}
    C6 = {(r, s): [[sum(m3c(r, i, j, k) * m3c(s, i, j, l)
                        for i in range(3) for j in range(3))
                    for l in range(3)] for k in range(3)] for (r, s) in T2}
    D7 = {(r, s): [sum(m3c(r, i, j, k) * m2c(s, i, j)
                       for i in range(3) for j in range(3))
                   for k in range(3)] for r in range(NR) for s in range(NR)}

    f = 0

    def emit(v):
        nonlocal f
        o_ref[f] = v
        f += 1

    for r in range(NR):
        emit(m0c(r))
    for (r, s) in T2:
        emit(sum(m1c(r, i) * m1c(s, i) for i in range(3)))
    for (r, s) in T2:
        emit(sum(m2c(r, i, j) * m2c(s, i, j) for i in range(3) for j in range(3)))
    for (r, s) in T2:
        emit(sum(m3c(r, i, j, k) * m3c(s, i, j, k)
                 for i in range(3) for j in range(3) for k in range(3)))
    for (r, s, t) in T3:
        emit(sum(A2[(r, s)][j][k] * m2c(t, j, k)
                 for j in range(3) for k in range(3)))
    for (r, s) in T2:
        for t in range(NR):
            emit(sum(B5[(r, t)][j] * m1c(s, j) for j in range(3)))
    for (r, s) in T2:
        for t in range(NR):
            emit(sum(C6[(r, s)][k][l] * m2c(t, k, l)
                     for k in range(3) for l in range(3)))
    for r in range(NR):
        for s in range(NR):
            for t in range(NR):
                emit(sum(D7[(r, s)][k] * m1c(t, k) for k in range(3)))
    assert f == NF


def kernel(R, Z, neighbor_idx, embeddings):
    idx0 = neighbor_idx[0]
    idx1 = neighbor_idx[1]
    pad = EP - E
    gi = jnp.concatenate([idx0, jnp.zeros((pad,), jnp.int32)])
    gj = jnp.concatenate([idx1, jnp.ones((pad,), jnp.int32)])
    seg = jnp.concatenate([idx1, jnp.full((pad,), AP - 1, jnp.int32)])
    rx = jnp.asarray(R[:, 0])
    ry = jnp.asarray(R[:, 1])
    rz = jnp.asarray(R[:, 2])
    emb2 = jnp.pad(embeddings.reshape(S * S, NR * NB),
                   ((0, 0), (0, WE - NR * NB)))
    zero_rows = jnp.zeros((_ROWS_PER_TILE, W), jnp.float32)

    partial = _edge_kernel()(gi, gj, seg, rx, ry, rz, Z, emb2, zero_rows)
    p4 = partial.reshape(NC, AP, W).transpose(0, 2, 1).reshape(NC, W, AP // 128, 128)

    feats = pl.pallas_call(
        _contract_body,
        grid=(AP // 1024,),
        in_specs=[pl.BlockSpec((NC, W, 8, 128), lambda i: (0, 0, i, 0))],
        out_specs=pl.BlockSpec((NF, 8, 128), lambda i: (0, i, 0)),
        out_shape=jax.ShapeDtypeStruct((NF, AP // 128, 128), jnp.float32),
    )(p4)
    return feats.reshape(NF, AP).T[:A]
